# baseline (device time: 94074 ns/iter reference)
import os

import jax
import jax.numpy as jnp
from jax import lax
from jax.experimental import pallas as pl
from jax.experimental.pallas import tpu as pltpu

N_DEV = 16
D_MODEL = 512
D_HIDDEN = 4096
CHUNK = D_HIDDEN // N_DEV
HQ = 64
DH = 64
H_PER = 4
SQ = 256
BLK = 64

F32 = jnp.float32
BF16 = jnp.bfloat16

_COMM_ONLY = os.environ.get("KERNEL_COMM_ONLY") == "1"
_COMPUTE_ONLY = os.environ.get("KERNEL_COMPUTE_ONLY") == "1"


def _fused(x_bf, wq_sh, wo_sh, k_bf, v_bf):
    n_b = x_bf.shape[0]

    def body(x_ref, wq_ref, wo_ref, k_ref, v_ref, out_ref,
             wq_buf, wo_buf, ctx_buf,
             zq_s, zq_r, zo_s, zo_r,
             pxq_s, pxq_r, pxo_s, pxo_r,
             pyq_s, pyq_r, pyo_s, pyo_r):
        my = lax.axis_index("i")
        my_q = lax.rem(my, 4)
        my_z4 = my - my_q
        xq = my_q + 1 - 2 * lax.rem(my_q, 2)
        yq = 3 - my_q
        dq = 3 - xq
        x_nb = my_z4 + xq
        y_nb = my_z4 + yq
        diag = my_z4 + dq

        qi = lax.broadcasted_iota(jnp.int32, (SQ, SQ), 0) // BLK
        kj = lax.broadcasted_iota(jnp.int32, (SQ, SQ), 1) // BLK
        neg = jnp.where(qi == kj, 0.0, -1e9).astype(F32)

        peers = [lax.rem(my + 4 * d, N_DEV) for d in (1, 2, 3)] + [x_nb, y_nb]
        barrier_sem = pltpu.get_barrier_semaphore()
        for nbr in peers:
            pl.semaphore_signal(
                barrier_sem, inc=1,
                device_id=(nbr,), device_id_type=pl.DeviceIdType.MESH,
            )
        pl.semaphore_wait(barrier_sem, len(peers))

        wq_buf[:, pl.ds(my * CHUNK, CHUNK)] = wq_ref[...]
        wo_buf[pl.ds(my * CHUNK, CHUNK), :] = wo_ref[...]

        def send_chunk(origin, target, sq_send, sq_recv, so_send, so_recv):
            rq = pltpu.make_async_remote_copy(
                src_ref=wq_buf.at[:, pl.ds(origin * CHUNK, CHUNK)],
                dst_ref=wq_buf.at[:, pl.ds(origin * CHUNK, CHUNK)],
                send_sem=sq_send, recv_sem=sq_recv,
                device_id=(target,), device_id_type=pl.DeviceIdType.MESH,
            )
            ro = pltpu.make_async_remote_copy(
                src_ref=wo_buf.at[pl.ds(origin * CHUNK, CHUNK), :],
                dst_ref=wo_buf.at[pl.ds(origin * CHUNK, CHUNK), :],
                send_sem=so_send, recv_sem=so_recv,
                device_id=(target,), device_id_type=pl.DeviceIdType.MESH,
            )
            rq.start()
            ro.start()
            return [rq, ro]

        def wait_chunk(origin, sq_recv, so_recv):
            rq = pltpu.make_async_remote_copy(
                src_ref=wq_buf.at[:, pl.ds(origin * CHUNK, CHUNK)],
                dst_ref=wq_buf.at[:, pl.ds(origin * CHUNK, CHUNK)],
                send_sem=sq_recv, recv_sem=sq_recv,
                device_id=(x_nb,), device_id_type=pl.DeviceIdType.MESH,
            )
            ro = pltpu.make_async_remote_copy(
                src_ref=wo_buf.at[pl.ds(origin * CHUNK, CHUNK), :],
                dst_ref=wo_buf.at[pl.ds(origin * CHUNK, CHUNK), :],
                send_sem=so_recv, recv_sem=so_recv,
                device_id=(x_nb,), device_id_type=pl.DeviceIdType.MESH,
            )
            rq.wait_recv()
            ro.wait_recv()

        def process_chunk(j, first=False):
            for b in range(n_b):
                q_c = lax.dot_general(
                    x_ref[b], wq_buf[:, pl.ds(j * CHUNK, CHUNK)],
                    (((1,), (0,)), ((), ())),
                    preferred_element_type=F32,
                ).astype(BF16)
                for hh in range(H_PER):
                    head = j * H_PER + hh
                    q_h = q_c[:, hh * DH:(hh + 1) * DH]
                    k_h = k_ref[b, head]
                    s = lax.dot_general(
                        q_h, k_h, (((1,), (1,)), ((), ())),
                        preferred_element_type=F32,
                    ) * 0.125 + neg
                    m = s.max(axis=-1, keepdims=True)
                    w = jnp.exp(s - m)
                    w = (w / w.sum(axis=-1, keepdims=True)).astype(BF16)
                    c = lax.dot_general(
                        w, v_ref[b, head], (((1,), (0,)), ((), ())),
                        preferred_element_type=F32,
                    )
                    ctx_buf[:, pl.ds(hh * DH, DH)] = c.astype(BF16)
                contrib = lax.dot_general(
                    ctx_buf[...], wo_buf[pl.ds(j * CHUNK, CHUNK), :],
                    (((1,), (0,)), ((), ())),
                    preferred_element_type=F32,
                )
                if first:
                    out_ref[b] = contrib
                else:
                    out_ref[b] = out_ref[b] + contrib

        def process(j, first=False):
            if not _COMM_ONLY:
                process_chunk(j, first=first)
            elif first:
                process_chunk(j, first=True)

        if _COMPUTE_ONLY:
            process_chunk(my, first=True)
            for _ in range(N_DEV - 1):
                process_chunk(my)
            return

        sends = []
        for d in (1, 2, 3):
            mate = lax.rem(my + 4 * d, N_DEV)
            sends += send_chunk(my, mate, zq_s.at[d], zq_r.at[d],
                                zo_s.at[d], zo_r.at[d])
        sends += send_chunk(my, x_nb, pxq_s.at[0, 0], pxq_r.at[0, 0],
                            pxo_s.at[0, 0], pxo_r.at[0, 0])
        sends += send_chunk(my, y_nb, pyq_s.at[0, 0], pyq_r.at[0, 0],
                            pyo_s.at[0, 0], pyo_r.at[0, 0])
        process(my, first=True)

        wait_chunk(x_nb, pxq_r.at[0, 0], pxo_r.at[0, 0])
        process(x_nb)
        wait_chunk(y_nb, pyq_r.at[0, 0], pyo_r.at[0, 0])
        sends += send_chunk(y_nb, x_nb, pxq_s.at[1, 0], pxq_r.at[1, 0],
                            pxo_s.at[1, 0], pxo_r.at[1, 0])
        process(y_nb)
        wait_chunk(diag, pxq_r.at[1, 0], pxo_r.at[1, 0])
        process(diag)

        for d in (1, 2, 3):
            ca = lax.rem(my + 4 * d, N_DEV)
            wait_chunk(ca, zq_r.at[4 - d], zo_r.at[4 - d])
            sends += send_chunk(ca, x_nb, pxq_s.at[0, d], pxq_r.at[0, d],
                                pxo_s.at[0, d], pxo_r.at[0, d])
            sends += send_chunk(ca, y_nb, pyq_s.at[0, d], pyq_r.at[0, d],
                                pyo_s.at[0, d], pyo_r.at[0, d])
            process(ca)
            cb = lax.rem(x_nb + 4 * d, N_DEV)
            wait_chunk(cb, pxq_r.at[0, d], pxo_r.at[0, d])
            if d % 2 == 1:
                sends += send_chunk(cb, y_nb, pyq_s.at[1, d], pyq_r.at[1, d],
                                    pyo_s.at[1, d], pyo_r.at[1, d])
            process(cb)
            cc = lax.rem(y_nb + 4 * d, N_DEV)
            wait_chunk(cc, pyq_r.at[0, d], pyo_r.at[0, d])
            if d % 2 == 0:
                sends += send_chunk(cc, x_nb, pxq_s.at[1, d], pxq_r.at[1, d],
                                    pxo_s.at[1, d], pxo_r.at[1, d])
            process(cc)
            cd = lax.rem(diag + 4 * d, N_DEV)
            if d % 2 == 0:
                wait_chunk(cd, pxq_r.at[1, d], pxo_r.at[1, d])
            else:
                wait_chunk(cd, pyq_r.at[1, d], pyo_r.at[1, d])
            process(cd)

        for r in sends:
            r.wait_send()

    return pl.pallas_call(
        body,
        out_shape=jax.ShapeDtypeStruct((n_b, SQ, D_MODEL), F32),
        in_specs=[pl.BlockSpec(memory_space=pltpu.VMEM)] * 5,
        out_specs=pl.BlockSpec(memory_space=pltpu.VMEM),
        scratch_shapes=[
            pltpu.VMEM((D_MODEL, D_HIDDEN), BF16),
            pltpu.VMEM((D_HIDDEN, D_MODEL), BF16),
            pltpu.VMEM((SQ, CHUNK), BF16),
            pltpu.SemaphoreType.DMA((4,)),
            pltpu.SemaphoreType.DMA((4,)),
            pltpu.SemaphoreType.DMA((4,)),
            pltpu.SemaphoreType.DMA((4,)),
            pltpu.SemaphoreType.DMA((2, 4)),
            pltpu.SemaphoreType.DMA((2, 4)),
            pltpu.SemaphoreType.DMA((2, 4)),
            pltpu.SemaphoreType.DMA((2, 4)),
            pltpu.SemaphoreType.DMA((2, 4)),
            pltpu.SemaphoreType.DMA((2, 4)),
            pltpu.SemaphoreType.DMA((2, 4)),
            pltpu.SemaphoreType.DMA((2, 4)),
        ],
        compiler_params=pltpu.CompilerParams(collective_id=0),
    )(x_bf, wq_sh, wo_sh, k_bf, v_bf)


def kernel(x, Wq, K_ext, V_ext, Wo):
    my = lax.axis_index("i")
    b = x.shape[0]

    K_loc = lax.dynamic_slice_in_dim(K_ext, my * b, b, axis=0)
    V_loc = lax.dynamic_slice_in_dim(V_ext, my * b, b, axis=0)
    k_bf = K_loc.transpose(0, 2, 1, 3).astype(BF16)
    v_bf = V_loc.transpose(0, 2, 1, 3).astype(BF16)

    return _fused(
        x.astype(BF16),
        Wq.astype(BF16),
        Wo.astype(BF16),
        k_bf,
        v_bf,
    )
